# Initial kernel scaffold; baseline (speedup 1.0000x reference)
import jax
import jax.numpy as jnp
from jax.experimental import pallas as pl


def kernel(x, edge_index, W_msg, b_msg, att_msg):
    raise NotImplementedError("write your pallas kernel here")


# speedup vs baseline: 1.0806x; 1.0806x over previous
import functools

import jax
import jax.numpy as jnp
from jax.experimental import pallas as pl
from jax.experimental.pallas import tpu as pltpu

N = 10000
E = 320000
C = 128


def _dense_body(x_ref, w_ref, b_ref, att_ref, y_ref, ela_ref):
    y = jnp.dot(x_ref[...], w_ref[...], preferred_element_type=jnp.float32)
    y = y + b_ref[...][None, :]
    y_ref[...] = y
    la = jnp.dot(y, att_ref[...].reshape(C, 1),
                 preferred_element_type=jnp.float32)
    la = jnp.where(la >= 0, la, 0.2 * la)
    ela_ref[...] = jnp.exp(la)


@jax.jit
def _dense(x, W_msg, b_msg, att_v):
    return pl.pallas_call(
        _dense_body,
        out_shape=(
            jax.ShapeDtypeStruct((N, C), jnp.float32),
            jax.ShapeDtypeStruct((N, 1), jnp.float32),
        ),
    )(x, W_msg, b_msg, att_v)


def kernel(x, edge_index, W_msg, b_msg, att_msg):
    src = edge_index[0]
    dst = edge_index[1]
    att_v = att_msg.reshape(C)
    y, ela2 = _dense(x, W_msg, b_msg, att_v)
    ela = ela2.reshape(N)

    g = ela[src]
    denom = jax.ops.segment_sum(g, dst, num_segments=N)
    cnt = jax.ops.segment_sum(jnp.ones((E,), jnp.float32), dst, num_segments=N)
    w = g / (denom[dst] + 1e-16)
    weighted = y[src] * w[:, None]
    s = jax.ops.segment_sum(weighted, dst, num_segments=N)
    mx = jax.ops.segment_max(weighted, dst, num_segments=N)
    mx = jnp.where(jnp.isfinite(mx), mx, 0.0)
    mn = jax.ops.segment_min(weighted, dst, num_segments=N)
    mn = jnp.where(jnp.isfinite(mn), mn, 0.0)
    mean = s / jnp.maximum(cnt, 1.0)[:, None]
    out = jnp.concatenate([mean, mx, s, mn], axis=-1)
    return out + jnp.tile(x, (1, 4))


# speedup vs baseline: 1.9584x; 1.8122x over previous
import functools

import jax
import jax.numpy as jnp
from jax import lax
from jax.experimental import pallas as pl
from jax.experimental.pallas import tpu as pltpu
from jax.experimental.pallas import tpu_sc as plsc

N = 10000
E = 320000
C = 128

_NC, _NS, _L = 2, 16, 16
_NW = _NC * _NS
_EPW = E // _NW
_NBF = _EPW // 128
_REM = _EPW - _NBF * 128

_MESH = plsc.VectorSubcoreMesh(
    core_axis_name="c", subcore_axis_name="s", num_cores=_NC, num_subcores=_NS)


def _dense_body(x_ref, w_ref, b_ref, att_ref, y_ref, ela_ref):
    y = jnp.dot(x_ref[...], w_ref[...], preferred_element_type=jnp.float32)
    y = y + b_ref[...][None, :]
    y_ref[...] = y
    la = jnp.dot(y, att_ref[...].reshape(C, 1),
                 preferred_element_type=jnp.float32)
    la = jnp.where(la >= 0, la, 0.2 * la)
    ela_ref[...] = jnp.exp(la)


def _dense(x, W_msg, b_msg, att_v):
    return pl.pallas_call(
        _dense_body,
        out_shape=(
            jax.ShapeDtypeStruct((N, C), jnp.float32),
            jax.ShapeDtypeStruct((N, 1), jnp.float32),
        ),
    )(x, W_msg, b_msg, att_v)


def _stats_body(src_h, dst_h, ela_h, zeros_h, g_h, den_h, cnt_h,
                ela_v, src_v, dst_v, val_v, ones_v,
                src16_v, dst16_v, val16_v, den_sh, cnt_sh):
    cid = lax.axis_index("c")
    sid = lax.axis_index("s")
    wid = sid * _NC + cid
    base = wid * _EPW

    pltpu.sync_copy(ela_h, ela_v)
    for k in range(8):
        ones_v[pl.ds(16 * k, 16)] = jnp.full((16,), 1.0, jnp.float32)

    @pl.when(sid == 0)
    def _():
        pltpu.sync_copy(zeros_h, den_sh)
        pltpu.sync_copy(zeros_h, cnt_sh)

    plsc.subcore_barrier()

    def body(i, carry):
        off = base + i * 128
        pltpu.sync_copy(src_h.at[pl.ds(off, 128)], src_v)
        pltpu.sync_copy(dst_h.at[pl.ds(off, 128)], dst_v)
        for k in range(8):
            s16 = src_v[pl.ds(16 * k, 16)]
            val_v[pl.ds(16 * k, 16)] = plsc.load_gather(ela_v, [s16])
        pltpu.sync_copy(val_v, g_h.at[pl.ds(off, 128)])
        pltpu.sync_copy(val_v, den_sh.at[dst_v], add=True)
        pltpu.sync_copy(ones_v, cnt_sh.at[dst_v], add=True)
        return carry

    lax.fori_loop(0, _NBF, body, 0)

    offr = base + _NBF * 128
    pltpu.sync_copy(src_h.at[pl.ds(offr, _REM)], src16_v)
    pltpu.sync_copy(dst_h.at[pl.ds(offr, _REM)], dst16_v)
    val16_v[...] = plsc.load_gather(ela_v, [src16_v[...]])
    pltpu.sync_copy(val16_v, g_h.at[pl.ds(offr, _REM)])
    pltpu.sync_copy(val16_v, den_sh.at[dst16_v], add=True)
    pltpu.sync_copy(ones_v.at[pl.ds(0, _REM)], cnt_sh.at[dst16_v], add=True)

    plsc.subcore_barrier()

    @pl.when(sid == 0)
    def _():
        pltpu.sync_copy(den_sh, den_h.at[cid])
        pltpu.sync_copy(cnt_sh, cnt_h.at[cid])


_stats = pl.kernel(
    _stats_body,
    out_type=(
        jax.ShapeDtypeStruct((E,), jnp.float32),
        jax.ShapeDtypeStruct((_NC, N), jnp.float32),
        jax.ShapeDtypeStruct((_NC, N), jnp.float32),
    ),
    mesh=_MESH,
    compiler_params=pltpu.CompilerParams(needs_layout_passes=False, use_tc_tiling_on_sc=False),
    scratch_types=[
        pltpu.VMEM((N,), jnp.float32),
        pltpu.VMEM((128,), jnp.int32),
        pltpu.VMEM((128,), jnp.int32),
        pltpu.VMEM((128,), jnp.float32),
        pltpu.VMEM((128,), jnp.float32),
        pltpu.VMEM((16,), jnp.int32),
        pltpu.VMEM((16,), jnp.int32),
        pltpu.VMEM((16,), jnp.float32),
        pltpu.VMEM_SHARED((N,), jnp.float32),
        pltpu.VMEM_SHARED((N,), jnp.float32),
    ],
)


_NT = 640
_NTT = _NT * _NS
_SB = 800
_NCHUNK = E // _SB
_BS = 32

_IOTA = None


def _rows_body(dst_h, src_h, g_h, y0_h, y1_h, den_h, s_h, mx_h, mn_h,
               acc_s, acc_mx, acc_mn, winv_v, dst_b, srcs_b, gs_b,
               dm_b, sm_b, gm_b, rows_v, sem, semr):
    cid = lax.axis_index("c")
    sid = lax.axis_index("s")
    nbase = _NT * sid
    iota = lax.iota(jnp.int32, 16)
    cols = [iota + 16 * c for c in range(4)]
    zeros16 = jnp.zeros((16,), jnp.float32)
    ninf16 = jnp.full((16,), -jnp.inf, jnp.float32)
    pinf16 = jnp.full((16,), jnp.inf, jnp.float32)

    def init_body(r, carry):
        rv = lax.broadcast(r, (16,))
        for c in range(4):
            plsc.store_scatter(acc_s, [rv, cols[c]], zeros16)
            plsc.store_scatter(acc_mx, [rv, cols[c]], ninf16)
            plsc.store_scatter(acc_mn, [rv, cols[c]], pinf16)
        return carry
    lax.fori_loop(0, _NT, init_body, 0)

    def zb_body(i, carry):
        z16 = jnp.zeros((16,), jnp.int32)
        plsc.store_scatter(sm_b, [iota + 16 * i], z16)
        plsc.store_scatter(dm_b, [iota + 16 * i], z16)
        plsc.store_scatter(gm_b, [iota + 16 * i], zeros16)
        return carry
    lax.fori_loop(0, (_SB + 16) // 16, zb_body, 0)

    pltpu.sync_copy(den_h.at[0, pl.ds(10 * sid, 10)], rows_v.at[pl.ds(0, 10)])
    pltpu.sync_copy(den_h.at[1, pl.ds(10 * sid, 10)], rows_v.at[pl.ds(10, 10)])
    for t in range(_NT // 16):
        r, cc = t // 4, (t % 4) * 16
        dtot = rows_v[r, pl.ds(cc, 16)] + rows_v[10 + r, pl.ds(cc, 16)]
        winv_v[pl.ds(16 * t, 16)] = 1.0 / (dtot + 1e-16)

    def chunk_body(i, carry):
        off = i * _SB
        cp1 = pltpu.async_copy(dst_h.at[pl.ds(off, _SB)], dst_b, sem)
        cp2 = pltpu.async_copy(src_h.at[pl.ds(off, _SB)], srcs_b, sem)
        cp3 = pltpu.async_copy(g_h.at[pl.ds(off, _SB)], gs_b, sem)
        cp1.wait(); cp2.wait(); cp3.wait()

        cursor = jnp.int32(0)
        for k in range(_SB // 16):
            d16 = dst_b[pl.ds(16 * k, 16)]
            dl = d16 - lax.broadcast(nbase, (16,))
            mk = (dl.astype(jnp.uint32) < jnp.uint32(_NT))
            plsc.store_compressed(dm_b.at[pl.ds(cursor, 16)], dl, mask=mk)
            plsc.store_compressed(sm_b.at[pl.ds(cursor, 16)],
                                  srcs_b[pl.ds(16 * k, 16)], mask=mk)
            plsc.store_compressed(gm_b.at[pl.ds(cursor, 16)],
                                  gs_b[pl.ds(16 * k, 16)], mask=mk)
            cursor = cursor + jnp.sum(mk.astype(jnp.int32))
        m = cursor

        def w_body(t, carry):
            dl16 = dm_b[pl.ds(16 * t, 16)]
            wv = gm_b[pl.ds(16 * t, 16)] * plsc.load_gather(winv_v, [dl16])
            gm_b[pl.ds(16 * t, 16)] = wv
            return carry
        lax.fori_loop(0, (m + 15) >> 4, w_body, 0)

        def batch_body(b, carry):
            boff = b * _BS

            @pl.when(cid == 0)
            def _():
                pltpu.async_copy(
                    y0_h.at[sm_b.at[pl.ds(boff, _BS)]], rows_v, semr).wait()

            @pl.when(cid == 1)
            def _():
                pltpu.async_copy(
                    y1_h.at[sm_b.at[pl.ds(boff, _BS)]], rows_v, semr).wait()

            mv = lax.broadcast(m, (16,))
            for j in range(_BS):
                ei = boff + j
                eiv = lax.broadcast(ei, (16,))
                mk = eiv < mv
                d16 = plsc.load_gather(dm_b, [eiv], mask=mk)
                w16 = plsc.load_gather(gm_b, [eiv], mask=mk)
                for c in range(4):
                    rowc = plsc.load_gather(rows_v, [lax.broadcast(j, (16,)),
                                                     cols[c]])
                    wr = w16 * rowc
                    plsc.addupdate_scatter(acc_s, [d16, cols[c]], wr, mask=mk)
                    cur = plsc.load_gather(acc_mx, [d16, cols[c]], mask=mk)
                    plsc.store_scatter(acc_mx, [d16, cols[c]],
                                       jnp.maximum(cur, wr), mask=mk)
                    cur2 = plsc.load_gather(acc_mn, [d16, cols[c]], mask=mk)
                    plsc.store_scatter(acc_mn, [d16, cols[c]],
                                       jnp.minimum(cur2, wr), mask=mk)
            return carry

        lax.fori_loop(0, (m + _BS - 1) // _BS, batch_body, 0)
        return carry

    lax.fori_loop(0, _NCHUNK, chunk_body, 0)

    pltpu.sync_copy(acc_s, s_h.at[cid, pl.ds(nbase, _NT)])
    pltpu.sync_copy(acc_mx, mx_h.at[cid, pl.ds(nbase, _NT)])
    pltpu.sync_copy(acc_mn, mn_h.at[cid, pl.ds(nbase, _NT)])


_rows = pl.kernel(
    _rows_body,
    out_type=(
        jax.ShapeDtypeStruct((_NC, _NTT, 64), jnp.float32),
        jax.ShapeDtypeStruct((_NC, _NTT, 64), jnp.float32),
        jax.ShapeDtypeStruct((_NC, _NTT, 64), jnp.float32),
    ),
    mesh=_MESH,
    compiler_params=pltpu.CompilerParams(needs_layout_passes=False, use_tc_tiling_on_sc=False),
    scratch_types=[
        pltpu.VMEM((_NT, 64), jnp.float32),
        pltpu.VMEM((_NT, 64), jnp.float32),
        pltpu.VMEM((_NT, 64), jnp.float32),
        pltpu.VMEM((_NT,), jnp.float32),
        pltpu.VMEM((_SB,), jnp.int32),
        pltpu.VMEM((_SB,), jnp.int32),
        pltpu.VMEM((_SB,), jnp.float32),
        pltpu.VMEM((_SB + 16,), jnp.int32),
        pltpu.VMEM((_SB + 16,), jnp.int32),
        pltpu.VMEM((_SB + 16,), jnp.float32),
        pltpu.VMEM((_BS, 64), jnp.float32),
        pltpu.SemaphoreType.DMA,
        pltpu.SemaphoreType.DMA,
    ],
)


def kernel(x, edge_index, W_msg, b_msg, att_msg):
    src = edge_index[0]
    dst = edge_index[1]
    att_v = att_msg.reshape(C)
    y, ela2 = _dense(x, W_msg, b_msg, att_v)
    ela = ela2.reshape(N)
    zeros_n = jnp.zeros((N,), jnp.float32)

    g, den_p, cnt_p = _stats(src, dst, ela, zeros_n)
    cnt = (cnt_p[0] + cnt_p[1])[:, None]

    y0 = y[:, :64]
    y1 = y[:, 64:]
    den_pad = jnp.concatenate(
        [den_p, jnp.ones((_NC, _NTT - N), jnp.float32)], axis=1)
    s3, mx3, mn3 = _rows(dst, src, g, y0, y1,
                         den_pad.reshape(_NC, _NTT // 64, 64))
    s = jnp.concatenate([s3[0, :N], s3[1, :N]], axis=-1)
    mx = jnp.concatenate([mx3[0, :N], mx3[1, :N]], axis=-1)
    mn = jnp.concatenate([mn3[0, :N], mn3[1, :N]], axis=-1)
    has = cnt > 0
    mx = jnp.where(has, mx, 0.0)
    mn = jnp.where(has, mn, 0.0)
    mean = s / jnp.maximum(cnt, 1.0)
    out = jnp.concatenate([mean, mx, s, mn], axis=-1)
    return out + jnp.tile(x, (1, 4))
